# Initial kernel scaffold; baseline (speedup 1.0000x reference)
import jax
import jax.numpy as jnp
from jax.experimental import pallas as pl


def kernel(A, X, W, b):
    raise NotImplementedError("write your pallas kernel here")


# speedup vs baseline: 2.6795x; 2.6795x over previous
import jax
import jax.numpy as jnp
from jax.experimental import pallas as pl
from jax.experimental.pallas import tpu as pltpu

_NQ = 128
_CBS = 512


def _body(intra_ref, scores_ref, a_ref, x_ref, newA_ref, newX_ref, idx_ref,
          g_ref):
    ns = intra_ref.shape[-1]
    n = scores_ref.shape[-1]
    kc = ns // 2
    m = kc + _NQ
    c = pl.program_id(1)

    @pl.when(c == 0)
    def _select():
        it_row = intra_ref[0]
        s_row = scores_ref[0]
        it_col = it_row.reshape(ns, 1)
        j_col = jax.lax.broadcasted_iota(jnp.int32, (ns, 1), 0)
        j_row = jax.lax.broadcasted_iota(jnp.int32, (1, ns), 1)

        CH = 384
        rank_chunks = []
        for c0 in range(0, ns, CH):
            it_i = jax.lax.slice(it_row, (0, c0), (1, c0 + CH))
            i_row = jax.lax.broadcasted_iota(jnp.int32, (1, CH), 1) + c0
            less = it_col < it_i
            tie = (it_col == it_i) & (j_col < i_row)
            mask = (less | tie).astype(jnp.int32)
            rank_chunks.append(jnp.sum(mask, axis=0, keepdims=True))
        rank_row = jnp.concatenate(rank_chunks, axis=1)

        s_supp_row = jax.lax.slice(s_row, (0, 0), (1, ns))
        PCH = 192
        idx_chunks, val_chunks = [], []
        for p0 in range(0, kc, PCH):
            p_col = jax.lax.broadcasted_iota(jnp.int32, (PCH, 1), 0) + p0
            onehot = rank_row == p_col
            idx_chunks.append(jnp.sum(
                jnp.where(onehot, j_row, 0), axis=1, keepdims=True))
            val_chunks.append(jnp.sum(
                jnp.where(onehot, s_supp_row, 0.0), axis=1, keepdims=True))
        q_iota = jax.lax.broadcasted_iota(jnp.int32, (_NQ, 1), 0) + ns
        s_col = s_row.reshape(n, 1)
        idx_col = jnp.concatenate(idx_chunks + [q_iota], axis=0)
        val_col = jnp.concatenate(
            val_chunks + [jax.lax.slice(s_col, (ns, 0), (n, 1))], axis=0)
        idx_ref[0] = idx_col.reshape(1, m)

        jn_row = jax.lax.broadcasted_iota(jnp.int32, (1, n), 1)
        RCH = 272
        for r0 in range(0, m, RCH):
            idx_c = jax.lax.slice(idx_col, (r0, 0), (r0 + RCH, 1))
            val_c = jax.lax.slice(val_col, (r0, 0), (r0 + RCH, 1))
            g_c = (idx_c == jn_row).astype(jnp.float32)
            g_ref[pl.ds(r0, RCH), :] = g_c
            newX_ref[0, pl.ds(r0, RCH), :] = jax.lax.dot_general(
                g_c * val_c, x_ref[0], (((1,), (0,)), ((), ())),
                preferred_element_type=jnp.float32)

    g_full = g_ref[...]
    rows_c = jax.lax.dot_general(
        g_full, a_ref[0], (((1,), (0,)), ((), ())),
        preferred_element_type=jnp.float32)
    g_c = g_ref[:, pl.ds(c * _CBS, _CBS)]
    contrib = jax.lax.dot_general(
        rows_c, g_c, (((1,), (1,)), ((), ())),
        preferred_element_type=jnp.float32)

    @pl.when(c == 0)
    def _init():
        newA_ref[0] = contrib

    @pl.when(c > 0)
    def _acc():
        newA_ref[0] += contrib


def kernel(A, X, W, b):
    B, N, D = X.shape
    ns = N - _NQ
    kc = ns // 2
    m = kc + _NQ
    scores = jax.nn.sigmoid(jnp.squeeze(X @ W + b, -1) / 100.0)
    supp = scores[:, :ns]
    intra = supp - jnp.mean(supp, axis=1, keepdims=True)

    newA, newX, idx3 = pl.pallas_call(
        _body,
        grid=(B, N // _CBS),
        in_specs=[
            pl.BlockSpec((1, 1, ns), lambda b_, c_: (b_, 0, 0)),
            pl.BlockSpec((1, 1, N), lambda b_, c_: (b_, 0, 0)),
            pl.BlockSpec((1, N, _CBS), lambda b_, c_: (b_, 0, c_)),
            pl.BlockSpec((1, N, D), lambda b_, c_: (b_, 0, 0)),
        ],
        out_specs=[
            pl.BlockSpec((1, m, m), lambda b_, c_: (b_, 0, 0)),
            pl.BlockSpec((1, m, D), lambda b_, c_: (b_, 0, 0)),
            pl.BlockSpec((1, 1, m), lambda b_, c_: (b_, 0, 0)),
        ],
        out_shape=[
            jax.ShapeDtypeStruct((B, m, m), jnp.float32),
            jax.ShapeDtypeStruct((B, m, D), jnp.float32),
            jax.ShapeDtypeStruct((B, 1, m), jnp.int32),
        ],
        scratch_shapes=[pltpu.VMEM((m, N), jnp.float32)],
        compiler_params=pltpu.CompilerParams(
            dimension_semantics=("arbitrary", "arbitrary")),
    )(intra.reshape(B, 1, ns), scores.reshape(B, 1, N), A, X)
    return newA, newX, idx3.reshape(B, m)
